# baseline (speedup 1.0000x reference)
import jax
import jax.numpy as jnp
from jax import lax
from jax.experimental import pallas as pl


def _fused_kernel(adj_ref, alpha_ref, uadj_ref, uops_ref, adj_out_ref, ops_out_ref):
    a = adj_ref[...]
    u = uadj_ref[...]
    act = jax.nn.sigmoid(a - jnp.log(-jnp.log(u)))
    i = lax.broadcasted_iota(jnp.int32, a.shape, 1)
    j = lax.broadcasted_iota(jnp.int32, a.shape, 2)
    adj_out_ref[...] = jnp.where(j > i, act, 0.0)

    y = alpha_ref[...] - jnp.log(-jnp.log(uops_ref[...]))
    m = jnp.max(y, axis=-1, keepdims=True)
    e = jnp.exp(y - m)
    ops_out_ref[...] = e / jnp.sum(e, axis=-1, keepdims=True)


def kernel(adj_para, ops_alpha, u_adj, u_ops):
    B, N, _ = adj_para.shape
    OPS = ops_alpha.shape[-1]
    BB = 32

    grid = (B // BB,)
    adj_spec = pl.BlockSpec((BB, N, N), lambda b: (b, 0, 0))
    ops_spec = pl.BlockSpec((BB, N, OPS), lambda b: (b, 0, 0))

    return pl.pallas_call(
        _fused_kernel,
        grid=grid,
        in_specs=[adj_spec, ops_spec, adj_spec, ops_spec],
        out_specs=[adj_spec, ops_spec],
        out_shape=[
            jax.ShapeDtypeStruct((B, N, N), adj_para.dtype),
            jax.ShapeDtypeStruct((B, N, OPS), ops_alpha.dtype),
        ],
    )(adj_para, ops_alpha, u_adj, u_ops)


# speedup vs baseline: 2.5613x; 2.5613x over previous
import jax
import jax.numpy as jnp
from jax import lax
from jax.experimental import pallas as pl


def _fused_kernel(adj_ref, alpha_ref, uadj_ref, uops_ref, mask_ref, m_ref,
                  adj_out_ref, ops_out_ref):
    a = adj_ref[...]
    t = -jnp.log(uadj_ref[...])
    act = 1.0 / (1.0 + t * jnp.exp(-a))
    adj_out_ref[...] = act * mask_ref[...]

    e = jnp.exp(alpha_ref[...]) / (-jnp.log(uops_ref[...]))
    s = jnp.dot(e, m_ref[...], precision=lax.Precision.HIGHEST,
                preferred_element_type=jnp.float32)
    ops_out_ref[...] = e / s


def kernel(adj_para, ops_alpha, u_adj, u_ops):
    B, N, _ = adj_para.shape
    OPS = ops_alpha.shape[-1]
    LA = N * N
    LO = N * OPS
    BB = 64

    adj_para2 = adj_para.reshape(B, LA)
    u_adj2 = u_adj.reshape(B, LA)
    ops_alpha2 = ops_alpha.reshape(B, LO)
    u_ops2 = u_ops.reshape(B, LO)

    l = lax.iota(jnp.int32, LA)
    mask = ((l % N) > (l // N)).astype(jnp.float32).reshape(1, LA)
    g = lax.iota(jnp.int32, LO) // OPS
    m = (g[:, None] == g[None, :]).astype(jnp.float32)

    grid = (B // BB,)
    adj_spec = pl.BlockSpec((BB, LA), lambda b: (b, 0))
    ops_spec = pl.BlockSpec((BB, LO), lambda b: (b, 0))
    mask_spec = pl.BlockSpec((1, LA), lambda b: (0, 0))
    m_spec = pl.BlockSpec((LO, LO), lambda b: (0, 0))

    adj_out, ops_out = pl.pallas_call(
        _fused_kernel,
        grid=grid,
        in_specs=[adj_spec, ops_spec, adj_spec, ops_spec, mask_spec, m_spec],
        out_specs=[adj_spec, ops_spec],
        out_shape=[
            jax.ShapeDtypeStruct((B, LA), adj_para.dtype),
            jax.ShapeDtypeStruct((B, LO), ops_alpha.dtype),
        ],
    )(adj_para2, ops_alpha2, u_adj2, u_ops2, mask, m)

    return adj_out.reshape(B, N, N), ops_out.reshape(B, N, OPS)
